# baseline (speedup 1.0000x reference)
import functools

import jax
import jax.numpy as jnp
import numpy as np
from jax import lax
from jax.experimental import pallas as pl
from jax.experimental.pallas import tpu as pltpu
from jax.experimental.pallas import tpu_sc as plsc

_K = 100
_KP = 128
_B = 512
_D = 512
_F_SCORE = 1.2447
_LN_BETA = float(np.log(0.999))
_NC, _NS, _L = 2, 16, 16
_NW = _NC * _NS
_BPW = _B // _NW
_NCH = _D // _L

_mesh = plsc.VectorSubcoreMesh(
    core_axis_name="c", subcore_axis_name="s", num_cores=_NC, num_subcores=_NS)


def _wid():
    return lax.axis_index("c") * _NS + lax.axis_index("s")


def _lane_iota():
    return lax.broadcasted_iota(jnp.int32, (_L,), 0)


def _sdiv(a, b):
    va = jnp.zeros((_L,), jnp.float32) + a
    vb = jnp.zeros((_L,), jnp.float32) + b
    return (va / vb)[0]


def _bsum_body(logits_ref, lab_ref, out_ref):
    x = logits_ref[...]
    labv = lab_ref[...]
    m = jnp.max(x, axis=1, keepdims=True)
    e = jnp.exp(x - m)
    p = e / jnp.sum(e, axis=1, keepdims=True)
    log_p = jnp.maximum(jnp.log(p), -100.0)
    log_1mp = jnp.maximum(jnp.log(1.0 - p), -100.0)
    oh = lax.broadcasted_iota(jnp.int32, x.shape, 1) == labv
    row = (jnp.sum(jnp.where(oh, log_p - log_1mp, 0.0), axis=1, keepdims=True)
           + jnp.sum(log_1mp, axis=1, keepdims=True))
    out_ref[...] = -row


def _bsum_tc(logits, labels):
    out = pl.pallas_call(
        _bsum_body,
        out_shape=jax.ShapeDtypeStruct((_B, 1), jnp.float32),
    )(logits, labels.reshape(_B, 1))
    return out.reshape(_B)


_ABLK = 64


def _pa_body(feat_hbm, lab_hbm, out_s, fblk, acc4, lab_v):
    w = _wid()
    pltpu.sync_copy(lab_hbm, lab_v)
    zeros16 = jnp.zeros((_L,), jnp.float32)
    for r in range(4):
        for j in range(_NCH):
            acc4[r, pl.ds(j * _L, _L)] = zeros16

    def block(b, carry):
        pltpu.sync_copy(feat_hbm.at[pl.ds(b * _ABLK, _ABLK)], fblk)

        def cloop(c, c2):
            labc = lab_v[pl.ds(b * _ABLK + c * _L, _L)]
            for i in range(_L):
                lab = labc[i]

                @pl.when(lax.rem(lab, _NW) == w)
                def _():
                    r = lax.div(lab, _NW)

                    def ch(j, c3):
                        acc4[r, pl.ds(j * _L, _L)] = (
                            acc4[r, pl.ds(j * _L, _L)]
                            + fblk[c * _L + i, pl.ds(j * _L, _L)])
                        return c3
                    lax.fori_loop(0, _NCH, ch, 0)
            return c2
        lax.fori_loop(0, _ABLK // _L, cloop, 0)
        return carry
    lax.fori_loop(0, _B // _ABLK, block, 0)
    for m in range(4):
        pltpu.sync_copy(acc4.at[pl.ds(m, 1)], out_s.at[pl.ds(w + _NW * m, 1)])


_phase_a = functools.partial(
    pl.kernel,
    out_type=jax.ShapeDtypeStruct((_KP, _D), jnp.float32),
    mesh=_mesh,
    compiler_params=pltpu.CompilerParams(needs_layout_passes=False),
    scratch_types=[
        pltpu.VMEM((_ABLK, _D), jnp.float32),
        pltpu.VMEM((4, _D), jnp.float32),
        pltpu.VMEM((_B,), jnp.int32),
    ],
)(_pa_body)


def _pb_body(feat_hbm, lab_hbm, s_hbm, cnt_hbm, z_out,
             feat_v, idx_v, rows, cnt_v, zres_v, sem):
    base = _wid() * _BPW
    pltpu.sync_copy(lab_hbm.at[pl.ds(base, _BPW)], idx_v)
    pltpu.sync_copy(cnt_hbm, cnt_v.at[pl.ds(0, _K)])
    pltpu.sync_copy(feat_hbm.at[pl.ds(base, _BPW)], feat_v)
    pltpu.async_copy(s_hbm.at[idx_v], rows, sem).wait()
    idxreg = idx_v[...]
    cntreg = plsc.load_gather(cnt_v, [idxreg])
    invreg = 1.0 / cntreg
    lane = _lane_iota()
    zvec = jnp.zeros((_L,), jnp.float32)
    for i in range(_BPW):
        inv = invreg[i]

        def ch(j, acc):
            f = feat_v[i, pl.ds(j * _L, _L)]
            m = rows[i, pl.ds(j * _L, _L)] * inv
            return acc + jnp.where(f != 0.0, jnp.abs(f - m), 0.0)
        acc = lax.fori_loop(0, _NCH, ch, jnp.zeros((_L,), jnp.float32))
        zvec = jnp.where(lane == i, jnp.sum(acc), zvec)
    zres_v[...] = zvec
    pltpu.sync_copy(zres_v, z_out.at[pl.ds(base, _BPW)])


_phase_b = functools.partial(
    pl.kernel,
    out_type=jax.ShapeDtypeStruct((_B,), jnp.float32),
    mesh=_mesh,
    compiler_params=pltpu.CompilerParams(needs_layout_passes=False),
    scratch_types=[
        pltpu.VMEM((_BPW, _D), jnp.float32),
        pltpu.VMEM((_BPW,), jnp.int32),
        pltpu.VMEM((_BPW, _D), jnp.float32),
        pltpu.VMEM((_KP,), jnp.float32),
        pltpu.VMEM((_BPW,), jnp.float32),
        pltpu.SemaphoreType.DMA,
    ],
)(_pb_body)


def _sqrt16(x):
    xi = lax.bitcast_convert_type(x, jnp.int32)
    yi = jnp.int32(0x5F3759DF) - lax.shift_right_logical(xi, 1)
    y = lax.bitcast_convert_type(yi, jnp.float32)
    for _ in range(4):
        y = y * (1.5 - 0.5 * x * y * y)
    return x * y


def _pc_body(z_hbm, lab_hbm, cnt_hbm, bsum_hbm, loss_out,
             z_v, lab_v, cnt_v, bsum_v, zsum_v, zim_v, sb_v, w_v, loss_v,
             zsum_sm):
    @pl.when(_wid() == 0)
    def _():
        pltpu.sync_copy(z_hbm, z_v)
        pltpu.sync_copy(lab_hbm, lab_v)
        pltpu.sync_copy(cnt_hbm, cnt_v.at[pl.ds(0, _K)])
        pltpu.sync_copy(bsum_hbm, bsum_v)
        lane = _lane_iota()

        def zbin(c, carry):
            zsum_sm[c] = 0.0
            return carry
        lax.fori_loop(0, _KP, zbin, 0)

        def bins(c, carry):
            zc = z_v[pl.ds(c * _L, _L)]
            labc = lab_v[pl.ds(c * _L, _L)]
            for i in range(_L):
                lab = labc[i]
                zsum_sm[lab] = zsum_sm[lab] + zc[i]
            return carry
        lax.fori_loop(0, _B // _L, bins, 0)

        for q in range(_KP // _L):
            vec = jnp.zeros((_L,), jnp.float32)
            for i in range(_L):
                vec = jnp.where(lane == i, zsum_sm[q * _L + i], vec)
            zsum_v[pl.ds(q * _L, _L)] = vec

        zm_acc = jnp.zeros((_L,), jnp.float32)
        n_acc = jnp.zeros((_L,), jnp.float32)
        for q in range(_KP // _L):
            valid = (lane + q * _L) < _K
            cnt_c = jnp.where(valid, cnt_v[pl.ds(q * _L, _L)], 1.0)
            zim_c = zsum_v[pl.ds(q * _L, _L)] / cnt_c
            zim_v[pl.ds(q * _L, _L)] = zim_c
            zm_acc = zm_acc + jnp.where(valid, zim_c, 0.0)
            n_acc = n_acc + jnp.where(valid, cnt_c, 0.0)
        z_mean = jnp.sum(zm_acc) * (1.0 / _K)
        n_tot = jnp.sum(n_acc)

        def sswc(c, acc):
            zc = z_v[pl.ds(c * _L, _L)]
            labc = lab_v[pl.ds(c * _L, _L)]
            zimg = plsc.load_gather(zim_v, [labc])
            d = zc - zimg
            return acc + jnp.where(zc != 0.0, d * d, 0.0)
        ssw_acc = lax.fori_loop(0, _B // _L, sswc,
                                jnp.zeros((_L,), jnp.float32))
        ssw = _sdiv(jnp.sum(ssw_acc), n_tot - float(_K))

        ssb_acc = jnp.zeros((_L,), jnp.float32)
        for q in range(_KP // _L):
            valid = (lane + q * _L) < _K
            cnt_c = jnp.where(valid, cnt_v[pl.ds(q * _L, _L)], 1.0)
            dzm = zim_v[pl.ds(q * _L, _L)] - z_mean
            sbm = jnp.where(valid, dzm * dzm * cnt_c, 0.0)
            sb_v[pl.ds(q * _L, _L)] = sbm
            ssb_acc = ssb_acc + sbm
        ssb = jnp.sum(ssb_acc) * (1.0 / (_K - 1))

        a = z_mean * z_mean
        inv2a = _sdiv(1.0, 2.0 * a)
        ws_acc = jnp.zeros((_L,), jnp.float32)
        for q in range(_KP // _L):
            valid = (lane + q * _L) < _K
            zsum_c = zsum_v[pl.ds(q * _L, _L)]
            cnt_c = jnp.where(valid, cnt_v[pl.ds(q * _L, _L)], 1.0)
            sb_c = sb_v[pl.ds(q * _L, _L)]
            cq = _F_SCORE * ssw * float(_K - 1) - (ssb * float(_K - 1) - sb_c)
            bq = -(2.0 * z_mean * zsum_c + cq)
            d2 = bq * bq - 4.0 * a * (zsum_c * zsum_c)
            dok = d2 >= 0.0
            dq = _sqrt16(jnp.maximum(d2, 0.0))
            n_lb = jnp.abs((-bq - dq) * inv2a)
            n_ub = jnp.abs((-bq + dq) * inv2a)
            c1 = jnp.logical_and(dok, cnt_c < n_lb)
            c2 = jnp.logical_and(dok, cnt_c > n_ub)
            t = jnp.where(c1, 1.0 / (n_lb - cnt_c),
                          jnp.where(c2, 1.0 / (cnt_c - n_ub), 1.0))
            beta = jnp.exp(_LN_BETA * t)
            en = 1.0 - jnp.exp(_LN_BETA * t * cnt_c)
            wr = (1.0 - beta) / en
            wrm = jnp.where(valid, wr, 0.0)
            w_v[pl.ds(q * _L, _L)] = wrm
            ws_acc = ws_acc + wrm
        wsum = jnp.sum(ws_acc)

        def dotc(c, acc):
            labc = lab_v[pl.ds(c * _L, _L)]
            wg = plsc.load_gather(w_v, [labc])
            return acc + wg * bsum_v[pl.ds(c * _L, _L)]
        dot_acc = lax.fori_loop(0, _B // _L, dotc,
                                jnp.zeros((_L,), jnp.float32))
        loss = jnp.sum(dot_acc) * _sdiv(float(_K), wsum) * (1.0 / (_B * _K))
        loss_v[...] = jnp.zeros((_L,), jnp.float32) + loss
        pltpu.sync_copy(loss_v, loss_out)


_phase_c = functools.partial(
    pl.kernel,
    out_type=jax.ShapeDtypeStruct((_L,), jnp.float32),
    mesh=_mesh,
    compiler_params=pltpu.CompilerParams(needs_layout_passes=False),
    scratch_types=[
        pltpu.VMEM((_B,), jnp.float32),
        pltpu.VMEM((_B,), jnp.int32),
        pltpu.VMEM((_KP,), jnp.float32),
        pltpu.VMEM((_B,), jnp.float32),
        pltpu.VMEM((_KP,), jnp.float32),
        pltpu.VMEM((_KP,), jnp.float32),
        pltpu.VMEM((_KP,), jnp.float32),
        pltpu.VMEM((_KP,), jnp.float32),
        pltpu.VMEM((_L,), jnp.float32),
        pltpu.SMEM((_KP,), jnp.float32),
    ],
)(_pc_body)


def kernel(logits, labels, features, sample_num_per_cls):
    labels = labels.astype(jnp.int32)
    bsum = _bsum_tc(logits, labels)
    s_tab = _phase_a(features, labels)
    z = _phase_b(features, labels, s_tab, sample_num_per_cls)
    loss_vec = _phase_c(z, labels, sample_num_per_cls, bsum)
    return loss_vec[0]
